# baseline (speedup 1.0000x reference)
import functools

import jax
import jax.numpy as jnp
from jax.experimental import pallas as pl

N = 5000
NP = 5120
BI = 256
SCORE_THR = 0.3
IOU_THR = 0.5


def _nms_body(aT_ref, a8_ref, out_ref):
    i = pl.program_id(0)

    cx = aT_ref[0:1, :]
    cy = aT_ref[1:2, :]
    w = aT_ref[3:4, :]
    l = aT_ref[4:5, :]
    ry = aT_ref[6:7, :]
    xl = aT_ref[7:8, :]
    c = jnp.abs(jnp.cos(ry))
    s = jnp.abs(jnp.sin(ry))
    hx = 0.5 * (w * c + l * s)
    hy = 0.5 * (w * s + l * c)
    jx1 = cx - hx
    jy1 = cy - hy
    jx2 = cx + hx
    jy2 = cy + hy
    ja = (jx2 - jx1) * (jy2 - jy1)
    js = jax.nn.sigmoid(xl)
    jvalid = js > SCORE_THR
    jsa = jnp.where(jvalid, js, -1.0)

    blk = a8_ref[...]
    icx = blk[:, 0:1]
    icy = blk[:, 1:2]
    iw_ = blk[:, 3:4]
    il_ = blk[:, 4:5]
    iry = blk[:, 6:7]
    ic = jnp.abs(jnp.cos(iry))
    is_ = jnp.abs(jnp.sin(iry))
    ihx = 0.5 * (iw_ * ic + il_ * is_)
    ihy = 0.5 * (iw_ * is_ + il_ * ic)
    ix1 = icx - ihx
    iy1 = icy - ihy
    ix2 = icx + ihx
    iy2 = icy + ihy
    ia = (ix2 - ix1) * (iy2 - iy1)
    isc = jax.nn.sigmoid(blk[:, 7:8])
    ivalid = isc > SCORE_THR

    iw = jnp.maximum(jnp.minimum(ix2, jx2) - jnp.maximum(ix1, jx1), 0.0)
    ih = jnp.maximum(jnp.minimum(iy2, jy2) - jnp.maximum(iy1, jy1), 0.0)
    inter = iw * ih
    union = ia + ja - inter
    overl = inter * 2.0 > union

    jidx = jax.lax.broadcasted_iota(jnp.int32, (1, NP), 1)
    iidx = i * BI + jax.lax.broadcasted_iota(jnp.int32, (BI, 1), 0)
    higher = (jsa > isc) | ((jsa == isc) & (jidx < iidx))

    sup = jnp.any(higher & overl, axis=1, keepdims=True)
    keepf = jnp.where(ivalid & ~sup, 1.0, 0.0)

    lane = jax.lax.broadcasted_iota(jnp.int32, (BI, 8), 1)
    vals = jnp.where(lane == 7, isc, blk)
    out_ref[...] = vals * keepf


@jax.jit
def kernel(x, guided_anchors):
    pad = NP - N
    aT = jnp.concatenate(
        [guided_anchors.T, x.reshape(1, N)], axis=0)
    aT = jnp.pad(aT, ((0, 0), (0, pad)),
                 constant_values=-100.0)
    a8 = jnp.concatenate([guided_anchors, x.reshape(N, 1)], axis=1)
    a8 = jnp.pad(a8, ((0, pad), (0, 0)), constant_values=-100.0)

    out = pl.pallas_call(
        _nms_body,
        grid=(NP // BI,),
        in_specs=[
            pl.BlockSpec((8, NP), lambda i: (0, 0)),
            pl.BlockSpec((BI, 8), lambda i: (i, 0)),
        ],
        out_specs=pl.BlockSpec((BI, 8), lambda i: (i, 0)),
        out_shape=jax.ShapeDtypeStruct((NP, 8), jnp.float32),
    )(aT, a8)
    return out[:N]
